# baseline (speedup 1.0000x reference)
import functools

import jax
import jax.numpy as jnp
from jax import lax
from jax.experimental import pallas as pl
from jax.experimental.pallas import tpu as pltpu
from jax.experimental.pallas import tpu_sc as plsc

N_NODES = 100000
D = 128
NUM_TYPES = 64

_info = plsc.get_sparse_core_info()
NC, NS = _info.num_cores, _info.num_subcores
NW = NC * NS

MAIN = (N_NODES // (8 * NW)) * (8 * NW)
B_W = MAIN // NW
CHUNK = 312
N_CHUNKS = B_W // CHUNK
N_PAIRS = N_CHUNKS // 2
TAIL = N_NODES - MAIN
TAIL_STEP = 8
TAIL_W = TAIL // TAIL_STEP

_mesh = plsc.VectorSubcoreMesh(core_axis_name="c", subcore_axis_name="s")


@functools.partial(
    pl.kernel,
    mesh=_mesh,
    out_type=(
        jax.ShapeDtypeStruct((N_NODES, D), jnp.float32),
        jax.ShapeDtypeStruct((N_NODES, D), jnp.float32),
    ),
    scratch_types=[
        pltpu.VMEM((B_W,), jnp.int32),
        pltpu.VMEM((TAIL_STEP,), jnp.int32),
        pltpu.VMEM((CHUNK, D), jnp.float32),
        pltpu.VMEM((CHUNK, D), jnp.float32),
        pltpu.VMEM_SHARED((NUM_TYPES, D), jnp.float32),
        pltpu.SemaphoreType.DMA,
        pltpu.SemaphoreType.DMA,
        pltpu.SemaphoreType.DMA,
        pltpu.SemaphoreType.DMA,
    ],
)
def _embed(idx_hbm, table_hbm, out0_hbm, out1_hbm, idx_v, tidx_v,
           buf0, buf1, table_sh, g0, g1, s0, s1):
    wid = lax.axis_index("s") * NC + lax.axis_index("c")
    base = wid * B_W

    @pl.when(lax.axis_index("s") == 0)
    def _stage():
        pltpu.sync_copy(table_hbm, table_sh)
    plsc.subcore_barrier()

    pltpu.sync_copy(idx_hbm.at[pl.ds(base, B_W)], idx_v)

    bufs, gsem, ssem = (buf0, buf1), (g0, g1), (s0, s1)
    outs = (out0_hbm, out1_hbm)

    def gather_start(c, b):
        pltpu.async_copy(
            table_sh.at[idx_v.at[pl.ds(c * CHUNK, CHUNK)]], bufs[b], gsem[b]
        )

    def gather_wait(b):
        pltpu.make_async_copy(
            table_sh.at[idx_v.at[pl.ds(0, CHUNK)]], bufs[b], gsem[b]
        ).wait()

    def store_start(c, b):
        for o in outs:
            pltpu.async_copy(
                bufs[b], o.at[pl.ds(base + c * CHUNK, CHUNK)], ssem[b]
            )

    def store_wait(b):
        for o in outs:
            pltpu.make_async_copy(
                bufs[b], o.at[pl.ds(base, CHUNK)], ssem[b]
            ).wait()

    gather_start(0, 0)

    def body(i, carry):
        @pl.when(i > 0)
        def _():
            store_wait(1)
        gather_start(2 * i + 1, 1)
        gather_wait(0)
        store_start(2 * i, 0)

        @pl.when(i < N_PAIRS - 1)
        def _():
            store_wait(0)
            gather_start(2 * i + 2, 0)
        gather_wait(1)
        store_start(2 * i + 1, 1)
        return carry

    lax.fori_loop(0, N_PAIRS, body, 0)
    store_wait(0)
    store_wait(1)

    @pl.when(wid < TAIL_W)
    def _tail():
        tb = MAIN + wid * TAIL_STEP
        pltpu.sync_copy(idx_hbm.at[pl.ds(tb, TAIL_STEP)], tidx_v)
        pltpu.async_copy(
            table_sh.at[tidx_v], buf0.at[pl.ds(0, TAIL_STEP)], g0
        ).wait()
        for o in outs:
            pltpu.sync_copy(
                buf0.at[pl.ds(0, TAIL_STEP)], o.at[pl.ds(tb, TAIL_STEP)]
            )


def kernel(atom_types, embed_table):
    idx = atom_types.reshape(-1).astype(jnp.int32)
    out0, out1 = _embed(idx, embed_table)
    return (out0, out1)


# speedup vs baseline: 1.0352x; 1.0352x over previous
import functools

import jax
import jax.numpy as jnp
from jax import lax
from jax.experimental import pallas as pl
from jax.experimental.pallas import tpu as pltpu
from jax.experimental.pallas import tpu_sc as plsc

N_NODES = 100000
D = 128
NUM_TYPES = 64

_info = plsc.get_sparse_core_info()
NC, NS = _info.num_cores, _info.num_subcores
NW = NC * NS

MAIN = (N_NODES // (8 * NW)) * (8 * NW)
B_W = MAIN // NW
CHUNK = 104
N_CHUNKS = B_W // CHUNK
NBUF = 5
N_ROUNDS = N_CHUNKS // NBUF
TAIL = N_NODES - MAIN
TAIL_STEP = 8
TAIL_W = TAIL // TAIL_STEP

_mesh = plsc.VectorSubcoreMesh(core_axis_name="c", subcore_axis_name="s")


@functools.partial(
    pl.kernel,
    mesh=_mesh,
    out_type=(
        jax.ShapeDtypeStruct((N_NODES, D), jnp.float32),
        jax.ShapeDtypeStruct((N_NODES, D), jnp.float32),
    ),
    scratch_types=[
        pltpu.VMEM((B_W,), jnp.int32),
        pltpu.VMEM((TAIL_STEP,), jnp.int32),
    ] + [pltpu.VMEM((CHUNK, D), jnp.float32) for _ in range(NBUF)] + [
        pltpu.VMEM_SHARED((NUM_TYPES, D), jnp.float32),
    ] + [pltpu.SemaphoreType.DMA for _ in range(2 * NBUF)],
)
def _embed(idx_hbm, table_hbm, out0_hbm, out1_hbm, idx_v, tidx_v, *rest):
    bufs = rest[:NBUF]
    table_sh = rest[NBUF]
    gsem = rest[NBUF + 1:NBUF + 1 + NBUF]
    ssem = rest[NBUF + 1 + NBUF:]
    wid = lax.axis_index("s") * NC + lax.axis_index("c")
    base = wid * B_W

    @pl.when(lax.axis_index("s") == 0)
    def _stage():
        pltpu.sync_copy(table_hbm, table_sh)
    plsc.subcore_barrier()

    pltpu.sync_copy(idx_hbm.at[pl.ds(base, B_W)], idx_v)

    outs = (out0_hbm, out1_hbm)

    def gather_start(c, b):
        pltpu.async_copy(
            table_sh.at[idx_v.at[pl.ds(c * CHUNK, CHUNK)]], bufs[b], gsem[b]
        )

    def gather_wait(b):
        pltpu.make_async_copy(
            table_sh.at[idx_v.at[pl.ds(0, CHUNK)]], bufs[b], gsem[b]
        ).wait()

    def store_start(c, b):
        for o in outs:
            pltpu.async_copy(
                bufs[b], o.at[pl.ds(base + c * CHUNK, CHUNK)], ssem[b]
            )

    def store_wait(b):
        for o in outs:
            pltpu.make_async_copy(
                bufs[b], o.at[pl.ds(base, CHUNK)], ssem[b]
            ).wait()

    def body(r, carry):
        for b in range(NBUF):
            @pl.when(r > 0)
            def _(b=b):
                store_wait(b)
            gather_start(r * NBUF + b, b)
        for b in range(NBUF):
            gather_wait(b)
            store_start(r * NBUF + b, b)
        return carry

    lax.fori_loop(0, N_ROUNDS, body, 0)
    for b in range(NBUF):
        store_wait(b)

    @pl.when(wid < TAIL_W)
    def _tail():
        tb = MAIN + wid * TAIL_STEP
        pltpu.sync_copy(idx_hbm.at[pl.ds(tb, TAIL_STEP)], tidx_v)
        pltpu.async_copy(
            table_sh.at[tidx_v], bufs[0].at[pl.ds(0, TAIL_STEP)], gsem[0]
        ).wait()
        for o in outs:
            pltpu.sync_copy(
                bufs[0].at[pl.ds(0, TAIL_STEP)], o.at[pl.ds(tb, TAIL_STEP)]
            )


def kernel(atom_types, embed_table):
    idx = atom_types.reshape(-1).astype(jnp.int32)
    out0, out1 = _embed(idx, embed_table)
    return (out0, out1)
